# baseline (speedup 1.0000x reference)
import functools

import jax
import jax.numpy as jnp
from jax import lax
from jax.experimental import pallas as pl
from jax.experimental.pallas import tpu as pltpu
from jax.experimental.pallas import tpu_sc as plsc

EMBED = 64
B_TOT = 4096 * 200
NW = 32
B_PER_W = B_TOT // NW
CHUNK = 1024
N_CHUNKS = B_PER_W // CHUNK

_mesh = plsc.VectorSubcoreMesh(core_axis_name="c", subcore_axis_name="s")


@functools.partial(
    pl.kernel,
    mesh=_mesh,
    out_type=jax.ShapeDtypeStruct((B_TOT, EMBED), jnp.float32),
    scratch_types=[
        pltpu.VMEM((CHUNK,), jnp.int32),
        pltpu.VMEM((CHUNK, EMBED), jnp.float32),
        pltpu.SemaphoreType.DMA,
    ],
    compiler_params=pltpu.CompilerParams(use_tc_tiling_on_sc=False),
)
def _embed_gather(table_hbm, idx_hbm, out_hbm, idx_v, rows_v, sem):
    wid = lax.axis_index("s") * 2 + lax.axis_index("c")
    base = wid * B_PER_W

    def body(i, carry):
        off = base + i * CHUNK
        pltpu.sync_copy(idx_hbm.at[pl.ds(off, CHUNK)], idx_v)
        pltpu.async_copy(table_hbm.at[idx_v], rows_v, sem).wait()
        pltpu.sync_copy(rows_v, out_hbm.at[pl.ds(off, CHUNK)])
        return carry

    lax.fori_loop(0, N_CHUNKS, body, 0)


def kernel(sentence, table):
    idx = sentence.reshape(-1).astype(jnp.int32)
    out = _embed_gather(table, idx)
    return out.reshape(sentence.shape + (EMBED,))


# speedup vs baseline: 1.0192x; 1.0192x over previous
import functools

import jax
import jax.numpy as jnp
from jax import lax
from jax.experimental import pallas as pl
from jax.experimental.pallas import tpu as pltpu
from jax.experimental.pallas import tpu_sc as plsc

EMBED = 64
B_TOT = 4096 * 200
NW = 32
B_PER_W = B_TOT // NW
CHUNK = 512
N_CHUNKS = B_PER_W // CHUNK
NBUF = 2
N_ROUNDS = N_CHUNKS // NBUF

_mesh = plsc.VectorSubcoreMesh(core_axis_name="c", subcore_axis_name="s")


@functools.partial(
    pl.kernel,
    mesh=_mesh,
    out_type=jax.ShapeDtypeStruct((B_TOT, EMBED), jnp.float32),
    scratch_types=[
        pltpu.VMEM((NBUF, CHUNK), jnp.int32),
        pltpu.VMEM((NBUF, CHUNK, EMBED), jnp.float32),
    ]
    + [pltpu.SemaphoreType.DMA] * (3 * NBUF),
    compiler_params=pltpu.CompilerParams(use_tc_tiling_on_sc=False),
)
def _embed_gather(table_hbm, idx_hbm, out_hbm, idx_v, rows_v, *sems):
    i_sem = sems[0:NBUF]
    g_sem = sems[NBUF : 2 * NBUF]
    s_sem = sems[2 * NBUF : 3 * NBUF]

    wid = lax.axis_index("s") * 2 + lax.axis_index("c")
    base = wid * B_PER_W

    def idx_copy(chunk, b):
        return pltpu.make_async_copy(
            idx_hbm.at[pl.ds(base + chunk * CHUNK, CHUNK)], idx_v.at[b], i_sem[b]
        )

    def gather_copy(b):
        return pltpu.make_async_copy(table_hbm.at[idx_v.at[b]], rows_v.at[b], g_sem[b])

    def store_copy(chunk, b):
        return pltpu.make_async_copy(
            rows_v.at[b], out_hbm.at[pl.ds(base + chunk * CHUNK, CHUNK)], s_sem[b]
        )

    for b in range(NBUF):
        idx_copy(b, b).start()
    for b in range(NBUF):
        idx_copy(b, b).wait()
        gather_copy(b).start()

    def body(r, carry):
        for b in range(NBUF):
            g = r * NBUF + b
            gather_copy(b).wait()
            store_copy(g, b).start()
            idx_copy(g + NBUF, b).start()
            store_copy(g, b).wait()
            idx_copy(g + NBUF, b).wait()
            gather_copy(b).start()
        return carry

    lax.fori_loop(0, N_ROUNDS - 1, body, 0)

    last = (N_ROUNDS - 1) * NBUF
    for b in range(NBUF):
        gather_copy(b).wait()
        store_copy(last + b, b).start()
    for b in range(NBUF):
        store_copy(last + b, b).wait()


def kernel(sentence, table):
    idx = sentence.reshape(-1).astype(jnp.int32)
    out = _embed_gather(table, idx)
    return out.reshape(sentence.shape + (EMBED,))


# speedup vs baseline: 1.2417x; 1.2183x over previous
import functools

import jax
import jax.numpy as jnp
from jax import lax
from jax.experimental import pallas as pl
from jax.experimental.pallas import tpu as pltpu
from jax.experimental.pallas import tpu_sc as plsc

EMBED = 64
ROW = 128
B_TOT = 4096 * 200
NW = 32
B_PER_W = B_TOT // NW
CHUNK = 256
N_CHUNKS = B_PER_W // CHUNK
NBUF = 2
N_ROUNDS = N_CHUNKS // NBUF

_mesh = plsc.VectorSubcoreMesh(core_axis_name="c", subcore_axis_name="s")


@functools.partial(
    pl.kernel,
    mesh=_mesh,
    out_type=jax.ShapeDtypeStruct((B_TOT, ROW), jnp.float32),
    scratch_types=[pltpu.VMEM((CHUNK,), jnp.int32)] * NBUF
    + [pltpu.VMEM((NBUF, CHUNK, ROW), jnp.float32)]
    + [pltpu.SemaphoreType.DMA] * (3 * NBUF),
    compiler_params=pltpu.CompilerParams(use_tc_tiling_on_sc=True),
)
def _embed_gather(table_hbm, idx_hbm, out_hbm, *scratch):
    idx_v = scratch[0:NBUF]
    rows_v = scratch[NBUF]
    sems = scratch[NBUF + 1 :]
    i_sem = sems[0:NBUF]
    g_sem = sems[NBUF : 2 * NBUF]
    s_sem = sems[2 * NBUF : 3 * NBUF]

    wid = lax.axis_index("s") * 2 + lax.axis_index("c")
    base = wid * B_PER_W

    def idx_copy(chunk, b):
        return pltpu.make_async_copy(
            idx_hbm.at[pl.ds(base + chunk * CHUNK, CHUNK)], idx_v[b], i_sem[b]
        )

    def gather_copy(b):
        return pltpu.make_async_copy(table_hbm.at[idx_v[b]], rows_v.at[b], g_sem[b])

    def store_copy(chunk, b):
        return pltpu.make_async_copy(
            rows_v.at[b], out_hbm.at[pl.ds(base + chunk * CHUNK, CHUNK)], s_sem[b]
        )

    for b in range(NBUF):
        idx_copy(b, b).start()
    for b in range(NBUF):
        idx_copy(b, b).wait()
        gather_copy(b).start()

    def body(r, carry):
        for b in range(NBUF):
            g = r * NBUF + b
            gather_copy(b).wait()
            store_copy(g, b).start()
            idx_copy(g + NBUF, b).start()
            store_copy(g, b).wait()
            idx_copy(g + NBUF, b).wait()
            gather_copy(b).start()
        return carry

    lax.fori_loop(0, N_ROUNDS - 1, body, 0)

    last = (N_ROUNDS - 1) * NBUF
    for b in range(NBUF):
        gather_copy(b).wait()
        store_copy(last + b, b).start()
    for b in range(NBUF):
        store_copy(last + b, b).wait()


def kernel(sentence, table):
    idx = sentence.reshape(-1).astype(jnp.int32)
    t128 = jnp.pad(table, ((0, 0), (0, ROW - EMBED)))
    out = _embed_gather(t128, idx)
    return out[:, :EMBED].reshape(sentence.shape + (EMBED,))
